# baseline (speedup 1.0000x reference)
import functools

import jax
import jax.numpy as jnp
from jax import lax
from jax.experimental import pallas as pl
from jax.experimental.pallas import tpu as pltpu
from jax.experimental.pallas import tpu_sc as plsc

N = 10000
F = 128
HF = 64
E1 = 320000
E2 = 640000
NC = 2
NS = 16

CH = 64
NBS = 8
NBR = 4
WROW = CH * 16 // 128
UNIT = NS * CH * NBS
E1P = ((E1 + UNIT - 1) // UNIT) * UNIT
E2P = ((E2 + UNIT - 1) // UNIT) * UNIT
NC1 = (E1P // NS) // CH
NC2 = (E2P // NS) // CH


def _spmm_body(hsplit, s1, d1, w1, s2, d2, w2, zeros,
               o1, o2, table, acc, idx_s, dst_s, w_s, rows_s, semi, semg, sems):
  c = lax.axis_index("c")
  s = lax.axis_index("s")

  @pl.when((s == 0) & (c == 0))
  def _():
    pltpu.sync_copy(hsplit.at[0], table)
    pltpu.sync_copy(zeros, acc)

  @pl.when((s == 0) & (c == 1))
  def _():
    pltpu.sync_copy(hsplit.at[1], table)
    pltpu.sync_copy(zeros, acc)

  plsc.subcore_barrier()

  def process(src_hbm, dst_hbm, ew_hbm, base, cbase, n):
    def issue_idx(j, q):
      off = pl.multiple_of(base + j * CH, 8)
      pltpu.async_copy(src_hbm.at[pl.ds(off, CH)], idx_s.at[q], semi.at[q])
      pltpu.async_copy(dst_hbm.at[pl.ds(off, CH)], dst_s.at[q], semi.at[q])
      pltpu.async_copy(ew_hbm.at[cbase + j], w_s.at[q], semi.at[q])

    def wait_idx(q):
      pltpu.make_async_copy(src_hbm.at[pl.ds(base, CH)], idx_s.at[q], semi.at[q]).wait()
      pltpu.make_async_copy(dst_hbm.at[pl.ds(base, CH)], dst_s.at[q], semi.at[q]).wait()
      pltpu.make_async_copy(ew_hbm.at[cbase], w_s.at[q], semi.at[q]).wait()

    def issue_gather(q, r):
      pltpu.async_copy(table.at[idx_s.at[q]], rows_s.at[r], semg.at[r])

    def wait_gather(r):
      pltpu.make_async_copy(table.at[idx_s.at[0]], rows_s.at[r], semg.at[r]).wait()

    def issue_scatter(q, r):
      pltpu.async_copy(rows_s.at[r], acc.at[dst_s.at[q]], sems.at[r], add=True)

    def wait_scatter(r):
      pltpu.make_async_copy(rows_s.at[0], acc.at[dst_s.at[0]], sems.at[r]).wait()

    def scale(q, r):
      def sbody(wr, carry):
        ibase = wr * 8
        for k in range(8):
          wb = w_s[q, wr, pl.ds(16 * k, 16)]
          i = ibase + k
          for jj in range(HF // 16):
            rows_s[r, i, pl.ds(jj * 16, 16)] = rows_s[r, i, pl.ds(jj * 16, 16)] * wb
        return carry
      lax.fori_loop(0, WROW, sbody, 0)

    for k in range(5):
      issue_idx(k, k)
    for k in range(3):
      wait_idx(k)
      issue_gather(k, k)

    def outer(g, carry):
      for qq in range(NBS):
        j = g * NBS + qq
        q, r = qq, qq % NBR
        q3, r3 = (qq + 3) % NBS, (qq + 3) % NBR
        q5 = (qq + 5) % NBS

        @pl.when(j + 5 < n)
        def _():
          issue_idx(j + 5, q5)

        @pl.when(j + 3 < n)
        def _():
          @pl.when(j >= 1)
          def _():
            wait_scatter(r3)
          wait_idx(q3)
          issue_gather(q3, r3)

        wait_gather(r)
        scale(q, r)
        issue_scatter(q, r)
      return carry

    lax.fori_loop(0, n // NBS, outer, 0)
    for r in range(NBR):
      wait_scatter(r)

  process(s1, d1, w1, s * (E1P // NS), s * NC1, NC1)
  plsc.subcore_barrier()

  @pl.when((s == 0) & (c == 0))
  def _():
    pltpu.sync_copy(acc, o1.at[0])
    pltpu.sync_copy(zeros, acc)

  @pl.when((s == 0) & (c == 1))
  def _():
    pltpu.sync_copy(acc, o1.at[1])
    pltpu.sync_copy(zeros, acc)

  plsc.subcore_barrier()

  process(s2, d2, w2, s * (E2P // NS), s * NC2, NC2)
  plsc.subcore_barrier()

  @pl.when((s == 0) & (c == 0))
  def _():
    pltpu.sync_copy(acc, o2.at[0])

  @pl.when((s == 0) & (c == 1))
  def _():
    pltpu.sync_copy(acc, o2.at[1])


_spmm = functools.partial(
    pl.kernel,
    out_type=[jax.ShapeDtypeStruct((NC, N, HF), jnp.float32),
              jax.ShapeDtypeStruct((NC, N, HF), jnp.float32)],
    mesh=plsc.VectorSubcoreMesh(core_axis_name="c", subcore_axis_name="s"),
    scratch_types=[
        pltpu.VMEM_SHARED((N, HF), jnp.float32),
        pltpu.VMEM_SHARED((N, HF), jnp.float32),
        pltpu.VMEM((NBS, CH), jnp.int32),
        pltpu.VMEM((NBS, CH), jnp.int32),
        pltpu.VMEM((NBS, WROW, 128), jnp.float32),
        pltpu.VMEM((NBR, CH, HF), jnp.float32),
        pltpu.SemaphoreType.DMA((NBS,)),
        pltpu.SemaphoreType.DMA((NBR,)),
        pltpu.SemaphoreType.DMA((NBR,)),
    ],
)(_spmm_body)


_RB = 1000


def _mm_body(split_out, hs_ref, y1_ref, y2_ref, w_ref, b_ref, o_ref):
  res = b_ref[...]
  for m, ref in enumerate((hs_ref, y1_ref, y2_ref)):
    for p in range(2):
      res = res + lax.dot_general(
          ref[p], w_ref[m, p], (((1,), (0,)), ((), ())),
          preferred_element_type=jnp.float32,
          precision=lax.Precision.HIGHEST)
  if split_out:
    o_ref[0] = res[:, :HF]
    o_ref[1] = res[:, HF:]
  else:
    o_ref[...] = res


def _mm(hs, y1, y2, w, b, split_out):
  half_spec = pl.BlockSpec((2, _RB, HF), lambda i: (0, i, 0))
  if split_out:
    out_shape = jax.ShapeDtypeStruct((2, N, HF), jnp.float32)
    out_spec = half_spec
  else:
    out_shape = jax.ShapeDtypeStruct((N, F), jnp.float32)
    out_spec = pl.BlockSpec((_RB, F), lambda i: (i, 0))
  return pl.pallas_call(
      functools.partial(_mm_body, split_out),
      grid=(N // _RB,),
      in_specs=[
          half_spec, half_spec, half_spec,
          pl.BlockSpec((3, 2, HF, F), lambda i: (0, 0, 0, 0)),
          pl.BlockSpec((1, F), lambda i: (0, 0)),
      ],
      out_specs=out_spec,
      out_shape=out_shape,
  )(hs, y1, y2, w, b)


def kernel(x, edge_index, edge_weight, edge_index2, edge_weight2,
           ib1_ln_W, ib1_ln_b, ib1_c1_W, ib1_c1_b, ib1_c2_W, ib1_c2_b,
           ib2_ln_W, ib2_ln_b, ib2_c1_W, ib2_c1_b, ib2_c2_W, ib2_c2_b,
           ib3_ln_W, ib3_ln_b, ib3_c1_W, ib3_c1_b, ib3_c2_W, ib3_c2_b):
  s1 = jnp.pad(edge_index[0].astype(jnp.int32), (0, E1P - E1))
  d1 = jnp.pad(edge_index[1].astype(jnp.int32), (0, E1P - E1))
  s2 = jnp.pad(edge_index2[0].astype(jnp.int32), (0, E2P - E2))
  d2 = jnp.pad(edge_index2[1].astype(jnp.int32), (0, E2P - E2))
  w1 = jnp.pad(jnp.tile(edge_weight.astype(jnp.float32)[:, None], (1, 16)),
               ((0, E1P - E1), (0, 0))).reshape(E1P // CH, WROW, 128)
  w2 = jnp.pad(jnp.tile(edge_weight2.astype(jnp.float32)[:, None], (1, 16)),
               ((0, E2P - E2), (0, 0))).reshape(E2P // CH, WROW, 128)
  zeros = jnp.zeros((N, HF), jnp.float32)

  def wstack(lw, c1w, c2w):
    return jnp.stack([lw.reshape(2, HF, F), c1w.reshape(2, HF, F),
                      c2w.reshape(2, HF, F)])

  wb = [
      (wstack(ib1_ln_W, ib1_c1_W, ib1_c2_W),
       (ib1_ln_b + ib1_c1_b + ib1_c2_b)[None, :]),
      (wstack(ib2_ln_W, ib2_c1_W, ib2_c2_W),
       (ib2_ln_b + ib2_c1_b + ib2_c2_b)[None, :]),
      (wstack(ib3_ln_W, ib3_c1_W, ib3_c2_W),
       (ib3_ln_b + ib3_c1_b + ib3_c2_b)[None, :]),
  ]

  hs = jnp.stack([x[:, :HF], x[:, HF:]])
  out = None
  for blk in range(3):
    y1, y2 = _spmm(hs, s1, d1, w1, s2, d2, w2, zeros)
    out = _mm(hs, y1, y2, wb[blk][0], wb[blk][1], split_out=(blk < 2))
    hs = out
  return out


# speedup vs baseline: 1.4769x; 1.4769x over previous
import functools

import jax
import jax.numpy as jnp
import numpy as np
from jax import lax
from jax.experimental import pallas as pl
from jax.experimental.pallas import tpu as pltpu
from jax.experimental.pallas import tpu_sc as plsc

N = 10000
F = 128
HF = 64
E1 = 320000
E2 = 640000
NC = 2
NS = 16

CH = 64
NBS = 8
NBR = 4
WROW = CH * 16 // 128
UNIT = NS * CH * NBS
E1P = ((E1 + UNIT - 1) // UNIT) * UNIT
E2P = ((E2 + UNIT - 1) // UNIT) * UNIT
NC1 = (E1P // NS) // CH
NC2 = (E2P // NS) // CH


def _spmm_body(hsplit, s1, d1, w1, s2, d2, w2, zeros,
               o1, o2, table, acc, idx_s, dst_s, w_s, rows_s, semi, semg, sems):
  c = lax.axis_index("c")
  s = lax.axis_index("s")

  @pl.when((s == 0) & (c == 0))
  def _():
    pltpu.sync_copy(hsplit.at[0], table)
    pltpu.sync_copy(zeros, acc)

  @pl.when((s == 0) & (c == 1))
  def _():
    pltpu.sync_copy(hsplit.at[1], table)
    pltpu.sync_copy(zeros, acc)

  plsc.subcore_barrier()

  def process(src_hbm, dst_hbm, ew_hbm, base, n):
    def issue_idx(j, q):
      off = pl.multiple_of(base + j * CH, 8)
      pltpu.async_copy(src_hbm.at[pl.ds(off, CH)], idx_s.at[q], semi.at[q])
      pltpu.async_copy(dst_hbm.at[pl.ds(off, CH)], dst_s.at[q], semi.at[q])
      pltpu.async_copy(ew_hbm.at[pl.ds(off, CH)], w_s.at[q], semi.at[q])

    def wait_idx(q):
      pltpu.make_async_copy(src_hbm.at[pl.ds(base, CH)], idx_s.at[q], semi.at[q]).wait()
      pltpu.make_async_copy(dst_hbm.at[pl.ds(base, CH)], dst_s.at[q], semi.at[q]).wait()
      pltpu.make_async_copy(ew_hbm.at[pl.ds(base, CH)], w_s.at[q], semi.at[q]).wait()

    def issue_gather(q, r):
      pltpu.async_copy(table.at[idx_s.at[q]], rows_s.at[r], semg.at[r])

    def wait_gather(r):
      pltpu.make_async_copy(table.at[idx_s.at[0]], rows_s.at[r], semg.at[r]).wait()

    def issue_scatter(q, r):
      pltpu.async_copy(rows_s.at[r], acc.at[dst_s.at[q]], sems.at[r], add=True)

    def wait_scatter(r):
      pltpu.make_async_copy(rows_s.at[0], acc.at[dst_s.at[0]], sems.at[r]).wait()

    def scale(q, r):
      dnums = lax.GatherDimensionNumbers(
          offset_dims=(), collapsed_slice_dims=(0,), start_index_map=(0,))
      def sbody(g, carry):
        ibase = g * 16
        wv = w_s[q, pl.ds(ibase, 16)]
        for k in range(16):
          wb = lax.gather(wv, jnp.full((16, 1), k, jnp.int32), dnums, (1,),
                          mode=lax.GatherScatterMode.PROMISE_IN_BOUNDS)
          i = ibase + k
          for jj in range(HF // 16):
            rows_s[r, i, pl.ds(jj * 16, 16)] = rows_s[r, i, pl.ds(jj * 16, 16)] * wb
        return carry
      lax.fori_loop(0, CH // 16, sbody, 0)

    for k in range(5):
      issue_idx(k, k)
    for k in range(3):
      wait_idx(k)
      issue_gather(k, k)

    def outer(g, carry):
      for qq in range(NBS):
        j = g * NBS + qq
        q, r = qq, qq % NBR
        q3, r3 = (qq + 3) % NBS, (qq + 3) % NBR
        q5 = (qq + 5) % NBS

        @pl.when(j + 5 < n)
        def _():
          issue_idx(j + 5, q5)

        @pl.when(j + 3 < n)
        def _():
          @pl.when(j >= 1)
          def _():
            wait_scatter(r3)
          wait_idx(q3)
          issue_gather(q3, r3)

        wait_gather(r)
        scale(q, r)
        issue_scatter(q, r)
      return carry

    lax.fori_loop(0, n // NBS, outer, 0)
    for r in range(NBR):
      wait_scatter(r)

  process(s1, d1, w1, s * (E1P // NS), NC1)
  plsc.subcore_barrier()

  @pl.when((s == 0) & (c == 0))
  def _():
    pltpu.sync_copy(acc, o1.at[0])
    pltpu.sync_copy(zeros, acc)

  @pl.when((s == 0) & (c == 1))
  def _():
    pltpu.sync_copy(acc, o1.at[1])
    pltpu.sync_copy(zeros, acc)

  plsc.subcore_barrier()

  process(s2, d2, w2, s * (E2P // NS), NC2)
  plsc.subcore_barrier()

  @pl.when((s == 0) & (c == 0))
  def _():
    pltpu.sync_copy(acc, o2.at[0])

  @pl.when((s == 0) & (c == 1))
  def _():
    pltpu.sync_copy(acc, o2.at[1])


_spmm = functools.partial(
    pl.kernel,
    out_type=[jax.ShapeDtypeStruct((NC, N, HF), jnp.float32),
              jax.ShapeDtypeStruct((NC, N, HF), jnp.float32)],
    mesh=plsc.VectorSubcoreMesh(core_axis_name="c", subcore_axis_name="s"),
    scratch_types=[
        pltpu.VMEM_SHARED((N, HF), jnp.float32),
        pltpu.VMEM_SHARED((N, HF), jnp.float32),
        pltpu.VMEM((NBS, CH), jnp.int32),
        pltpu.VMEM((NBS, CH), jnp.int32),
        pltpu.VMEM((NBS, CH), jnp.float32),
        pltpu.VMEM((NBR, CH, HF), jnp.float32),
        pltpu.SemaphoreType.DMA((NBS,)),
        pltpu.SemaphoreType.DMA((NBR,)),
        pltpu.SemaphoreType.DMA((NBR,)),
    ],
)(_spmm_body)


_RB = 1000


def _mm_body(split_out, hs_ref, y1_ref, y2_ref, w_ref, b_ref, o_ref):
  res = b_ref[...]
  for m, ref in enumerate((hs_ref, y1_ref, y2_ref)):
    for p in range(2):
      res = res + lax.dot_general(
          ref[p], w_ref[m, p], (((1,), (0,)), ((), ())),
          preferred_element_type=jnp.float32,
          precision=lax.Precision.HIGHEST)
  if split_out:
    o_ref[0] = res[:, :HF]
    o_ref[1] = res[:, HF:]
  else:
    o_ref[...] = res


def _mm(hs, y1, y2, w, b, split_out):
  half_spec = pl.BlockSpec((2, _RB, HF), lambda i: (0, i, 0))
  if split_out:
    out_shape = jax.ShapeDtypeStruct((2, N, HF), jnp.float32)
    out_spec = half_spec
  else:
    out_shape = jax.ShapeDtypeStruct((N, F), jnp.float32)
    out_spec = pl.BlockSpec((_RB, F), lambda i: (i, 0))
  return pl.pallas_call(
      functools.partial(_mm_body, split_out),
      grid=(N // _RB,),
      in_specs=[
          half_spec, half_spec, half_spec,
          pl.BlockSpec((3, 2, HF, F), lambda i: (0, 0, 0, 0)),
          pl.BlockSpec((1, F), lambda i: (0, 0)),
      ],
      out_specs=out_spec,
      out_shape=out_shape,
  )(hs, y1, y2, w, b)


def kernel(x, edge_index, edge_weight, edge_index2, edge_weight2,
           ib1_ln_W, ib1_ln_b, ib1_c1_W, ib1_c1_b, ib1_c2_W, ib1_c2_b,
           ib2_ln_W, ib2_ln_b, ib2_c1_W, ib2_c1_b, ib2_c2_W, ib2_c2_b,
           ib3_ln_W, ib3_ln_b, ib3_c1_W, ib3_c1_b, ib3_c2_W, ib3_c2_b):
  s1 = jnp.pad(edge_index[0].astype(jnp.int32), (0, E1P - E1))
  d1 = jnp.pad(edge_index[1].astype(jnp.int32), (0, E1P - E1))
  s2 = jnp.pad(edge_index2[0].astype(jnp.int32), (0, E2P - E2))
  d2 = jnp.pad(edge_index2[1].astype(jnp.int32), (0, E2P - E2))
  w1 = jnp.pad(edge_weight.astype(jnp.float32), (0, E1P - E1))
  w2 = jnp.pad(edge_weight2.astype(jnp.float32), (0, E2P - E2))
  zeros = jnp.zeros((N, HF), jnp.float32)

  def wstack(lw, c1w, c2w):
    return jnp.stack([lw.reshape(2, HF, F), c1w.reshape(2, HF, F),
                      c2w.reshape(2, HF, F)])

  wb = [
      (wstack(ib1_ln_W, ib1_c1_W, ib1_c2_W),
       (ib1_ln_b + ib1_c1_b + ib1_c2_b)[None, :]),
      (wstack(ib2_ln_W, ib2_c1_W, ib2_c2_W),
       (ib2_ln_b + ib2_c1_b + ib2_c2_b)[None, :]),
      (wstack(ib3_ln_W, ib3_c1_W, ib3_c2_W),
       (ib3_ln_b + ib3_c1_b + ib3_c2_b)[None, :]),
  ]

  hs = jnp.stack([x[:, :HF], x[:, HF:]])
  out = None
  for blk in range(3):
    y1, y2 = _spmm(hs, s1, d1, w1, s2, d2, w2, zeros)
    out = _mm(hs, y1, y2, wb[blk][0], wb[blk][1], split_out=(blk < 2))
    hs = out
  return out


# speedup vs baseline: 1.4816x; 1.0032x over previous
import functools

import jax
import jax.numpy as jnp
from jax import lax
from jax.experimental import pallas as pl
from jax.experimental.pallas import tpu as pltpu
from jax.experimental.pallas import tpu_sc as plsc

N = 10000
F = 128
HF = 64
E1 = 320000
E2 = 640000
NC = 2
NS = 16

CH = 64
NBS = 8
NBR = 4
UNIT = NS * CH * NBS
E1P = ((E1 + UNIT - 1) // UNIT) * UNIT
E2P = ((E2 + UNIT - 1) // UNIT) * UNIT
NC1 = (E1P // NS) // CH
NC2 = (E2P // NS) // CH


def _spmm_body(hsplit, s1, d1, w1, s2, d2, w2, zeros,
               o1, o2, table, acc, idx_s, dst_s, w_s, rows_s, semi, semg, sems):
  c = lax.axis_index("c")
  s = lax.axis_index("s")

  @pl.when((s == 0) & (c == 0))
  def _():
    pltpu.sync_copy(hsplit.at[0], table)
    pltpu.sync_copy(zeros, acc)

  @pl.when((s == 0) & (c == 1))
  def _():
    pltpu.sync_copy(hsplit.at[1], table)
    pltpu.sync_copy(zeros, acc)

  plsc.subcore_barrier()

  def process(src_hbm, dst_hbm, ew_hbm, base, n):
    def issue_idx(j, q):
      off = pl.multiple_of(base + j * CH, 8)
      pltpu.async_copy(src_hbm.at[pl.ds(off, CH)], idx_s.at[q], semi.at[q])
      pltpu.async_copy(dst_hbm.at[pl.ds(off, CH)], dst_s.at[q], semi.at[q])
      pltpu.async_copy(ew_hbm.at[pl.ds(off, CH)], w_s.at[q], semi.at[q])

    def wait_idx(q):
      pltpu.make_async_copy(src_hbm.at[pl.ds(base, CH)], idx_s.at[q], semi.at[q]).wait()
      pltpu.make_async_copy(dst_hbm.at[pl.ds(base, CH)], dst_s.at[q], semi.at[q]).wait()
      pltpu.make_async_copy(ew_hbm.at[pl.ds(base, CH)], w_s.at[q], semi.at[q]).wait()

    def issue_gather(q, r):
      pltpu.async_copy(table.at[idx_s.at[q]], rows_s.at[r], semg.at[r])

    def wait_gather(r):
      pltpu.make_async_copy(table.at[idx_s.at[0]], rows_s.at[r], semg.at[r]).wait()

    def issue_scatter(q, r):
      pltpu.async_copy(rows_s.at[r], acc.at[dst_s.at[q]], sems.at[r], add=True)

    def wait_scatter(r):
      pltpu.make_async_copy(rows_s.at[0], acc.at[dst_s.at[0]], sems.at[r]).wait()

    def scale(q, r):
      dnums = lax.GatherDimensionNumbers(
          offset_dims=(), collapsed_slice_dims=(0,), start_index_map=(0,))
      def sbody(g, carry):
        ibase = g * 16
        wv = w_s[q, pl.ds(ibase, 16)]
        for k in range(16):
          wb = lax.gather(wv, jnp.full((16, 1), k, jnp.int32), dnums, (1,),
                          mode=lax.GatherScatterMode.PROMISE_IN_BOUNDS)
          i = ibase + k
          for jj in range(HF // 16):
            rows_s[r, i, pl.ds(jj * 16, 16)] = rows_s[r, i, pl.ds(jj * 16, 16)] * wb
        return carry
      lax.fori_loop(0, CH // 16, sbody, 0)

    for k in range(5):
      issue_idx(k, k)
    for k in range(3):
      wait_idx(k)
      issue_gather(k, k)

    def outer(g, carry):
      for qq in range(NBS):
        j = g * NBS + qq
        q, r = qq, qq % NBR
        q3, r3 = (qq + 3) % NBS, (qq + 3) % NBR
        q5 = (qq + 5) % NBS

        @pl.when(j + 5 < n)
        def _():
          issue_idx(j + 5, q5)

        @pl.when(j + 3 < n)
        def _():
          @pl.when(j >= 1)
          def _():
            wait_scatter(r3)
          wait_idx(q3)
          issue_gather(q3, r3)

        wait_gather(r)
        scale(q, r)
        issue_scatter(q, r)
      return carry

    lax.fori_loop(0, n // NBS, outer, 0)
    for r in range(NBR):
      wait_scatter(r)

  process(s1, d1, w1, s * (E1P // NS), NC1)
  plsc.subcore_barrier()

  @pl.when((s == 0) & (c == 0))
  def _():
    pltpu.sync_copy(acc, o1.at[0])
    pltpu.sync_copy(zeros, acc)

  @pl.when((s == 0) & (c == 1))
  def _():
    pltpu.sync_copy(acc, o1.at[1])
    pltpu.sync_copy(zeros, acc)

  plsc.subcore_barrier()

  process(s2, d2, w2, s * (E2P // NS), NC2)
  plsc.subcore_barrier()

  @pl.when((s == 0) & (c == 0))
  def _():
    pltpu.sync_copy(acc, o2.at[0])

  @pl.when((s == 0) & (c == 1))
  def _():
    pltpu.sync_copy(acc, o2.at[1])


_spmm = functools.partial(
    pl.kernel,
    out_type=[jax.ShapeDtypeStruct((NC, N, HF), jnp.float32),
              jax.ShapeDtypeStruct((NC, N, HF), jnp.float32)],
    mesh=plsc.VectorSubcoreMesh(core_axis_name="c", subcore_axis_name="s"),
    scratch_types=[
        pltpu.VMEM_SHARED((N, HF), jnp.float32),
        pltpu.VMEM_SHARED((N, HF), jnp.float32),
        pltpu.VMEM((NBS, CH), jnp.int32),
        pltpu.VMEM((NBS, CH), jnp.int32),
        pltpu.VMEM((NBS, CH), jnp.float32),
        pltpu.VMEM((NBR, CH, HF), jnp.float32),
        pltpu.SemaphoreType.DMA((NBS,)),
        pltpu.SemaphoreType.DMA((NBR,)),
        pltpu.SemaphoreType.DMA((NBR,)),
    ],
)(_spmm_body)


_RB = 1000


def _mm_body(split_out, hs_ref, y1_ref, y2_ref, w_ref, b_ref, o_ref):
  res = b_ref[...]
  for m, ref in enumerate((hs_ref, y1_ref, y2_ref)):
    for p in range(2):
      res = res + lax.dot_general(
          ref[p], w_ref[m, p], (((1,), (0,)), ((), ())),
          preferred_element_type=jnp.float32,
          precision=lax.Precision.HIGHEST)
  if split_out:
    o_ref[0] = res[:, :HF]
    o_ref[1] = res[:, HF:]
  else:
    o_ref[...] = res


def _mm(hs, y1, y2, w, b, split_out):
  half_spec = pl.BlockSpec((2, _RB, HF), lambda i: (0, i, 0))
  if split_out:
    out_shape = jax.ShapeDtypeStruct((2, N, HF), jnp.float32)
    out_spec = half_spec
  else:
    out_shape = jax.ShapeDtypeStruct((N, F), jnp.float32)
    out_spec = pl.BlockSpec((_RB, F), lambda i: (i, 0))
  return pl.pallas_call(
      functools.partial(_mm_body, split_out),
      grid=(N // _RB,),
      in_specs=[
          half_spec, half_spec, half_spec,
          pl.BlockSpec((3, 2, HF, F), lambda i: (0, 0, 0, 0)),
          pl.BlockSpec((1, F), lambda i: (0, 0)),
      ],
      out_specs=out_spec,
      out_shape=out_shape,
  )(hs, y1, y2, w, b)


def kernel(x, edge_index, edge_weight, edge_index2, edge_weight2,
           ib1_ln_W, ib1_ln_b, ib1_c1_W, ib1_c1_b, ib1_c2_W, ib1_c2_b,
           ib2_ln_W, ib2_ln_b, ib2_c1_W, ib2_c1_b, ib2_c2_W, ib2_c2_b,
           ib3_ln_W, ib3_ln_b, ib3_c1_W, ib3_c1_b, ib3_c2_W, ib3_c2_b):
  s1 = jnp.pad(edge_index[0].astype(jnp.int32), (0, E1P - E1))
  d1 = jnp.pad(edge_index[1].astype(jnp.int32), (0, E1P - E1))
  s2 = jnp.pad(edge_index2[0].astype(jnp.int32), (0, E2P - E2))
  d2 = jnp.pad(edge_index2[1].astype(jnp.int32), (0, E2P - E2))
  w1 = jnp.pad(edge_weight.astype(jnp.float32), (0, E1P - E1))
  w2 = jnp.pad(edge_weight2.astype(jnp.float32), (0, E2P - E2))
  zeros = jnp.zeros((N, HF), jnp.float32)

  def wstack(lw, c1w, c2w):
    return jnp.stack([lw.reshape(2, HF, F), c1w.reshape(2, HF, F),
                      c2w.reshape(2, HF, F)])

  wb = [
      (wstack(ib1_ln_W, ib1_c1_W, ib1_c2_W),
       (ib1_ln_b + ib1_c1_b + ib1_c2_b)[None, :]),
      (wstack(ib2_ln_W, ib2_c1_W, ib2_c2_W),
       (ib2_ln_b + ib2_c1_b + ib2_c2_b)[None, :]),
      (wstack(ib3_ln_W, ib3_c1_W, ib3_c2_W),
       (ib3_ln_b + ib3_c1_b + ib3_c2_b)[None, :]),
  ]

  hs = jnp.stack([x[:, :HF], x[:, HF:]])
  out = None
  for blk in range(3):
    y1, y2 = _spmm(hs, s1, d1, w1, s2, d2, w2, zeros)
    out = _mm(hs, y1, y2, wb[blk][0], wb[blk][1], split_out=(blk < 2))
    hs = out
  return out
